# Initial kernel scaffold; baseline (speedup 1.0000x reference)
import jax
import jax.numpy as jnp
from jax.experimental import pallas as pl


def kernel(x, pe):
    raise NotImplementedError("write your pallas kernel here")


# speedup vs baseline: 3.5421x; 3.5421x over previous
import functools

import jax
import jax.numpy as jnp
from jax import lax
from jax.experimental import pallas as pl
from jax.experimental.pallas import tpu as pltpu
from jax.experimental.pallas import tpu_sc as plsc

_D = 64
_CHUNK = 128
_NW = 32


@functools.partial(jax.jit, static_argnums=())
def _gather_rows(idx2d, table):
    chunks_total = idx2d.shape[0]
    chunks_per_w = chunks_total // _NW
    rows_per_w = chunks_per_w * _CHUNK
    total_rows = chunks_total * _CHUNK

    mesh = plsc.VectorSubcoreMesh(core_axis_name="c", subcore_axis_name="s")

    @functools.partial(
        pl.kernel,
        mesh=mesh,
        out_type=jax.ShapeDtypeStruct((total_rows, _D), jnp.float32),
        scratch_types=[
            pltpu.VMEM((chunks_per_w, _CHUNK), jnp.int32),
            pltpu.VMEM((_CHUNK, _D), jnp.float32),
            pltpu.SemaphoreType.DMA,
        ],
        compiler_params=pltpu.CompilerParams(use_tc_tiling_on_sc=False),
    )
    def body(idx_hbm, table_hbm, out_hbm, idx_v, rows_v, sem):
        wid = lax.axis_index("s") * 2 + lax.axis_index("c")
        pltpu.sync_copy(idx_hbm.at[pl.ds(wid * chunks_per_w, chunks_per_w)], idx_v)
        out_base = wid * rows_per_w

        def step(j, carry):
            pltpu.async_copy(table_hbm.at[idx_v.at[j]], rows_v, sem).wait()
            pltpu.sync_copy(rows_v, out_hbm.at[pl.ds(out_base + j * _CHUNK, _CHUNK)])
            return carry

        lax.fori_loop(0, chunks_per_w, step, 0)

    return body(idx2d, table)


def kernel(x, pe):
    b, l = x.shape
    idx2d = x.reshape(-1, _CHUNK)
    out = _gather_rows(idx2d, pe)
    return out.reshape(b, l, _D)


# speedup vs baseline: 4.2668x; 1.2046x over previous
import functools

import jax
import jax.numpy as jnp
from jax import lax
from jax.experimental import pallas as pl
from jax.experimental.pallas import tpu as pltpu
from jax.experimental.pallas import tpu_sc as plsc

_D = 64
_CHUNK = 128
_K = 4
_GROUP = _K * _CHUNK
_NW = 32


def _gather_rows(idx2d, table):
    chunks_total = idx2d.shape[0]
    chunks_per_w = chunks_total // _NW
    rows_per_w = chunks_per_w * _CHUNK
    total_rows = chunks_total * _CHUNK
    rounds = chunks_per_w // _K
    assert chunks_per_w % _K == 0 and rounds % 2 == 0 and rounds >= 4

    mesh = plsc.VectorSubcoreMesh(core_axis_name="c", subcore_axis_name="s")

    @functools.partial(
        pl.kernel,
        mesh=mesh,
        out_type=jax.ShapeDtypeStruct((total_rows, _D), jnp.float32),
        scratch_types=[
            pltpu.VMEM((chunks_per_w, _CHUNK), jnp.int32),
            pltpu.VMEM((_GROUP, _D), jnp.float32),
            pltpu.VMEM((_GROUP, _D), jnp.float32),
            pltpu.SemaphoreType.DMA,
            pltpu.SemaphoreType.DMA,
            pltpu.SemaphoreType.DMA,
            pltpu.SemaphoreType.DMA,
        ],
        compiler_params=pltpu.CompilerParams(use_tc_tiling_on_sc=False),
    )
    def body(idx_hbm, table_hbm, out_hbm, idx_v, rows0, rows1,
             gsem0, gsem1, ssem0, ssem1):
        wid = lax.axis_index("s") * 2 + lax.axis_index("c")
        pltpu.sync_copy(idx_hbm.at[pl.ds(wid * chunks_per_w, chunks_per_w)], idx_v)
        out_base = wid * rows_per_w

        def fire_gathers(r, grp, gsem):
            for b in range(_K):
                pltpu.async_copy(table_hbm.at[idx_v.at[r * _K + b]],
                                 grp.at[pl.ds(b * _CHUNK, _CHUNK)], gsem)

        def wait_gathers(r, grp, gsem):
            for b in range(_K):
                pltpu.make_async_copy(table_hbm.at[idx_v.at[r * _K + b]],
                                      grp.at[pl.ds(b * _CHUNK, _CHUNK)], gsem).wait()

        def fire_scatter(r, grp, ssem):
            pltpu.async_copy(grp, out_hbm.at[pl.ds(out_base + r * _GROUP, _GROUP)], ssem)

        def wait_scatter(r, grp, ssem):
            pltpu.make_async_copy(grp, out_hbm.at[pl.ds(out_base + r * _GROUP, _GROUP)],
                                  ssem).wait()

        fire_gathers(0, rows0, gsem0)
        fire_gathers(1, rows1, gsem1)
        wait_gathers(0, rows0, gsem0)
        fire_scatter(0, rows0, ssem0)

        @pl.loop(0, (rounds - 2) // 2)
        def _steady(i):
            r = 1 + 2 * i
            wait_scatter(r - 1, rows0, ssem0)
            fire_gathers(r + 1, rows0, gsem0)
            wait_gathers(r, rows1, gsem1)
            fire_scatter(r, rows1, ssem1)
            wait_scatter(r, rows1, ssem1)
            fire_gathers(r + 2, rows1, gsem1)
            wait_gathers(r + 1, rows0, gsem0)
            fire_scatter(r + 1, rows0, ssem0)

        r_last = rounds - 1
        wait_scatter(r_last - 1, rows0, ssem0)
        wait_gathers(r_last, rows1, gsem1)
        fire_scatter(r_last, rows1, ssem1)
        wait_scatter(r_last, rows1, ssem1)

    return body(idx2d, table)


def kernel(x, pe):
    b, l = x.shape
    idx2d = x.reshape(-1, _CHUNK)
    out = _gather_rows(idx2d, pe)
    return out.reshape(b, l, _D)
